# baseline (speedup 1.0000x reference)
import jax
import jax.numpy as jnp
from jax.experimental import pallas as pl
from jax.experimental.pallas import tpu as pltpu


def _copy_body(e_ref, eo_ref):
    eo_ref[...] = e_ref[...]


def kernel(x, edge_index, edge_attr):
    del edge_index
    e2 = edge_attr.reshape(10000, 256)
    e_out = pl.pallas_call(
        _copy_body,
        grid=(5,),
        in_specs=[pl.BlockSpec((2000, 256), lambda i: (i, 0))],
        out_specs=pl.BlockSpec((2000, 256), lambda i: (i, 0)),
        out_shape=jax.ShapeDtypeStruct((10000, 256), edge_attr.dtype),
    )(e2)
    return (x, e_out.reshape(edge_attr.shape))


# speedup vs baseline: 1.2439x; 1.2439x over previous
import jax
import jax.numpy as jnp
from jax.experimental import pallas as pl
from jax.experimental.pallas import tpu as pltpu

_GRID = 25
_XBLK = 400
_EBLK = 6400


def _copy_body(x_ref, e_ref, xo_ref, eo_ref):
    xo_ref[...] = x_ref[...]
    eo_ref[...] = e_ref[...]


def kernel(x, edge_index, edge_attr):
    del edge_index
    x_out, e_out = pl.pallas_call(
        _copy_body,
        grid=(_GRID,),
        in_specs=[
            pl.BlockSpec((_XBLK, 256), lambda i: (i, 0)),
            pl.BlockSpec((_EBLK, 16), lambda i: (i, 0)),
        ],
        out_specs=[
            pl.BlockSpec((_XBLK, 256), lambda i: (i, 0)),
            pl.BlockSpec((_EBLK, 16), lambda i: (i, 0)),
        ],
        out_shape=[
            jax.ShapeDtypeStruct(x.shape, x.dtype),
            jax.ShapeDtypeStruct(edge_attr.shape, edge_attr.dtype),
        ],
        compiler_params=pltpu.CompilerParams(
            dimension_semantics=("parallel",),
        ),
    )(x, edge_attr)
    return (x_out, e_out)


# speedup vs baseline: 1.2571x; 1.0106x over previous
import jax
import jax.numpy as jnp
from jax.experimental import pallas as pl
from jax.experimental.pallas import tpu as pltpu

_GRID = 10
_XBLK = 1000
_EBLK = 16000


def _copy_body(x_ref, e_ref, xo_ref, eo_ref):
    xo_ref[...] = x_ref[...]
    eo_ref[...] = e_ref[...]


def kernel(x, edge_index, edge_attr):
    del edge_index
    x_out, e_out = pl.pallas_call(
        _copy_body,
        grid=(_GRID,),
        in_specs=[
            pl.BlockSpec((_XBLK, 256), lambda i: (i, 0)),
            pl.BlockSpec((_EBLK, 16), lambda i: (i, 0)),
        ],
        out_specs=[
            pl.BlockSpec((_XBLK, 256), lambda i: (i, 0)),
            pl.BlockSpec((_EBLK, 16), lambda i: (i, 0)),
        ],
        out_shape=[
            jax.ShapeDtypeStruct(x.shape, x.dtype),
            jax.ShapeDtypeStruct(edge_attr.shape, edge_attr.dtype),
        ],
        compiler_params=pltpu.CompilerParams(
            dimension_semantics=("parallel",),
        ),
    )(x, edge_attr)
    return (x_out, e_out)
